# baseline (speedup 1.0000x reference)
import functools

import jax
import jax.numpy as jnp
from jax import lax
from jax.experimental import pallas as pl
from jax.experimental.pallas import tpu as pltpu
from jax.experimental.pallas import tpu_sc as plsc

N = 10000
NPAD = 10240
E = 320000
NC, NS = 2, 16
NW = NC * NS
CH = 64
EPAD = 327680
EPT = EPAD // NW
NCH = EPT // CH
RPT = NPAD // NS
RB = CH
RCH = RPT // RB


def _scatter_body(g_hbm, row_hbm, col_hbm, zeros_hbm, out_hbm,
                  ridx, cidx, bb0, bb1, fb0, fb1, acc, g0, g1, s0, s1):
    cid = lax.axis_index("c")
    sid = lax.axis_index("s")
    wid = sid * NC + cid
    pltpu.sync_copy(row_hbm.at[pl.ds(wid * NCH, NCH)], ridx)
    pltpu.sync_copy(col_hbm.at[pl.ds(wid * NCH, NCH)], cidx)
    pltpu.sync_copy(zeros_hbm, fb0)
    for k in range(RCH):
        pltpu.sync_copy(fb0, acc.at[pl.ds(sid * RPT + k * RB, RB)])
    plsc.subcore_barrier()

    bbs = (bb0, bb1)
    fbs = (fb0, fb1)
    gsem = (g0, g1)
    ssem = (s0, s1)
    d = fb0.shape[1]
    ngr = d // 32
    pltpu.async_copy(g_hbm.at[ridx.at[0]], bbs[0], gsem[0])
    niter = NCH // 2

    def step(i, carry):
        for k in range(2):
            j = 2 * i + k
            kn = 1 - k

            @pl.when(j + 1 < NCH)
            def _():
                pltpu.async_copy(g_hbm.at[ridx.at[j + 1]], bbs[kn], gsem[kn])

            pltpu.make_async_copy(g_hbm.at[ridx.at[j]], bbs[k], gsem[k]).wait()

            @pl.when(j >= 2)
            def _():
                pltpu.make_async_copy(
                    fbs[k], acc.at[cidx.at[j - 2]], ssem[k]).wait()

            def conv(r2, carry2):
                for u in range(2):
                    r = 2 * r2 + u
                    for c in range(ngr):
                        v = bbs[k][r, pl.ds(c * 32, 32)]
                        a, b = plsc.unpack(
                            v, format=plsc.PackFormat.INTERLEAVED)
                        fbs[k][r, pl.ds(c * 32, 16)] = a
                        fbs[k][r, pl.ds(c * 32 + 16, 16)] = b
                return carry2

            lax.fori_loop(0, CH // 2, conv, 0)
            pltpu.async_copy(fbs[k], acc.at[cidx.at[j]], ssem[k], add=True)
        return carry

    lax.fori_loop(0, niter, step, 0)
    pltpu.make_async_copy(fbs[0], acc.at[cidx.at[NCH - 2]], ssem[0]).wait()
    pltpu.make_async_copy(fbs[1], acc.at[cidx.at[NCH - 1]], ssem[1]).wait()
    plsc.subcore_barrier()
    for k in range(RCH):
        pltpu.sync_copy(acc.at[pl.ds(sid * RPT + k * RB, RB)], fb0)
        pltpu.sync_copy(fb0, out_hbm.at[cid, pl.ds(sid * RPT + k * RB, RB)])


def _deg_body(col_hbm, ones_hbm, zeros_hbm, out_hbm, cidx, ones_v, zbuf, acc,
              d0, d1, d2, d3):
    cid = lax.axis_index("c")
    sid = lax.axis_index("s")
    wid = sid * NC + cid
    pltpu.sync_copy(col_hbm.at[pl.ds(wid * NCH, NCH)], cidx)
    pltpu.sync_copy(ones_hbm, ones_v)
    pltpu.sync_copy(zeros_hbm, zbuf)
    pltpu.sync_copy(zbuf, acc.at[pl.ds(sid * RPT, RPT)])
    plsc.subcore_barrier()

    sems = (d0, d1, d2, d3)

    def step(i, carry):
        for k in range(4):
            j = 4 * i + k

            @pl.when(j >= 4)
            def _():
                pltpu.make_async_copy(
                    ones_v, acc.at[cidx.at[j - 4]], sems[k]).wait()

            pltpu.async_copy(ones_v, acc.at[cidx.at[j]], sems[k], add=True)
        return carry

    lax.fori_loop(0, NCH // 4, step, 0)
    for k in range(4):
        pltpu.make_async_copy(
            ones_v, acc.at[cidx.at[NCH - 4 + k]], sems[k]).wait()
    plsc.subcore_barrier()
    pltpu.sync_copy(acc.at[pl.ds(sid * RPT, RPT)], zbuf)
    pltpu.sync_copy(zbuf, out_hbm.at[cid, pl.ds(sid * RPT, RPT)])


@functools.lru_cache(maxsize=None)
def _make_deg():
    mesh = plsc.VectorSubcoreMesh(core_axis_name="c", subcore_axis_name="s")
    return pl.kernel(
        _deg_body,
        out_type=jax.ShapeDtypeStruct((NC, NPAD), jnp.float32),
        mesh=mesh,
        scratch_types=[
            pltpu.VMEM((NCH, CH), jnp.int32),
            pltpu.VMEM((CH,), jnp.float32),
            pltpu.VMEM((RPT,), jnp.float32),
            pltpu.VMEM_SHARED((NPAD,), jnp.float32),
        ] + [pltpu.SemaphoreType.DMA] * 4,
        compiler_params=pltpu.CompilerParams(use_tc_tiling_on_sc=False),
    )


@functools.lru_cache(maxsize=None)
def _make_scatter(d):
    mesh = plsc.VectorSubcoreMesh(core_axis_name="c", subcore_axis_name="s")
    return pl.kernel(
        _scatter_body,
        out_type=jax.ShapeDtypeStruct((NC, NPAD, d), jnp.float32),
        mesh=mesh,
        scratch_types=[
            pltpu.VMEM((NCH, CH), jnp.int32),
            pltpu.VMEM((NCH, CH), jnp.int32),
            pltpu.VMEM((CH, d), jnp.bfloat16),
            pltpu.VMEM((CH, d), jnp.bfloat16),
            pltpu.VMEM((CH, d), jnp.float32),
            pltpu.VMEM((CH, d), jnp.float32),
            pltpu.VMEM_SHARED((NPAD, d), jnp.float32),
        ] + [pltpu.SemaphoreType.DMA] * 4,
        compiler_params=pltpu.CompilerParams(use_tc_tiling_on_sc=False,
                                             needs_layout_passes=False),
    )


_BR = 1024
_NB = NPAD // _BR


def _tc_first_body(d0_ref, d1_ref, x_ref, w_ref, dis_ref, h_ref, g_ref):
    deg = d0_ref[...] + d1_ref[...] + 1.0
    dis = lax.rsqrt(deg)
    h = jnp.dot(x_ref[...], w_ref[...], preferred_element_type=jnp.float32)
    dis_ref[...] = dis
    h_ref[...] = h
    g_ref[...] = h * dis


def _tc_mid_body(a0_ref, a1_ref, hp_ref, dis_ref, b_ref, w_ref, h_ref, g_ref):
    dis = dis_ref[...]
    act = dis * (a0_ref[...] + a1_ref[...]) + (dis * dis) * hp_ref[...] + b_ref[...]
    act = jnp.maximum(act, 0.0)
    h = jnp.dot(act, w_ref[...], preferred_element_type=jnp.float32)
    h_ref[...] = h
    g_ref[...] = h * dis


def _tc_last_body(a0_ref, a1_ref, hp_ref, dis_ref, b_ref, out_ref):
    dis = dis_ref[...]
    out_ref[...] = (dis * (a0_ref[...] + a1_ref[...])
                    + (dis * dis) * hp_ref[...] + b_ref[...])


def _row_spec(d):
    return pl.BlockSpec((_BR, d), lambda i: (i, 0))


def _full_spec(r, c):
    return pl.BlockSpec((r, c), lambda i: (0, 0))


def _tc_first(deg0, deg1, x, w):
    din, dout = w.shape
    return pl.pallas_call(
        _tc_first_body,
        grid=(_NB,),
        in_specs=[_row_spec(1), _row_spec(1), _row_spec(din),
                  _full_spec(din, dout)],
        out_specs=[_row_spec(1), _row_spec(dout), _row_spec(dout)],
        out_shape=[jax.ShapeDtypeStruct((NPAD, 1), jnp.float32),
                   jax.ShapeDtypeStruct((NPAD, dout), jnp.float32),
                   jax.ShapeDtypeStruct((NPAD, dout), jnp.float32)],
    )(deg0, deg1, x, w)


def _tc_mid(a0, a1, hp, dis, b, w):
    din, dout = w.shape
    return pl.pallas_call(
        _tc_mid_body,
        grid=(_NB,),
        in_specs=[_row_spec(din), _row_spec(din), _row_spec(din),
                  _row_spec(1), _full_spec(1, din), _full_spec(din, dout)],
        out_specs=[_row_spec(dout), _row_spec(dout)],
        out_shape=[jax.ShapeDtypeStruct((NPAD, dout), jnp.float32),
                   jax.ShapeDtypeStruct((NPAD, dout), jnp.float32)],
    )(a0, a1, hp, dis, b, w)


def _tc_last(a0, a1, hp, dis, b):
    d = hp.shape[1]
    return pl.pallas_call(
        _tc_last_body,
        grid=(_NB,),
        in_specs=[_row_spec(d), _row_spec(d), _row_spec(d),
                  _row_spec(1), _full_spec(1, d)],
        out_specs=_row_spec(d),
        out_shape=jax.ShapeDtypeStruct((NPAD, d), jnp.float32),
    )(a0, a1, hp, dis, b)


def kernel(x, edge_index, W1, b1, W2, b2, W3, b3, W4, b4, W5, b5):
    rowp = jnp.pad(edge_index[0], (0, EPAD - E))
    colp = jnp.pad(edge_index[1], (0, EPAD - E), constant_values=N)
    row2 = rowp.reshape(EPAD // CH, CH)
    col2 = colp.reshape(EPAD // CH, CH)
    xp = jnp.pad(x, ((0, NPAD - N), (0, 0)))

    degp = _make_deg()(col2, jnp.ones((CH,), jnp.float32),
                       jnp.zeros((RPT,), jnp.float32))
    dis, h, g = _tc_first(degp[0][:, None], degp[1][:, None], xp, W1)

    w5p = jnp.pad(W5, ((0, 0), (0, 128 - W5.shape[1])))
    b5p = jnp.pad(b5, (0, 128 - b5.shape[0]))
    ws = [W2, W3, W4, w5p]
    bs = [b1, b2, b3, b4]
    z128 = jnp.zeros((RB, 128), jnp.float32)

    def prep(gv):
        p = gv.reshape(NPAD, 4, 2, 16).swapaxes(2, 3).reshape(NPAD, 128)
        return p.astype(jnp.bfloat16)

    for i in range(4):
        aggp = _make_scatter(128)(prep(g), row2, col2, z128)
        h, g = _tc_mid(aggp[0], aggp[1], h, dis, bs[i].reshape(1, -1), ws[i])
    aggp = _make_scatter(128)(prep(g), row2, col2, z128)
    out = _tc_last(aggp[0], aggp[1], h, dis, b5p.reshape(1, -1))
    return out[:N, :W5.shape[1]]


# speedup vs baseline: 1.1767x; 1.1767x over previous
import functools

import jax
import jax.numpy as jnp
import numpy as np
from jax import lax
from jax.experimental import pallas as pl
from jax.experimental.pallas import tpu as pltpu
from jax.experimental.pallas import tpu_sc as plsc

N = 10000
NPAD = 10240
E = 320000
EL = E + N
NC, NS = 2, 16
NW = NC * NS
CH = 64
EPAD = 331776
EPT = EPAD // NW
NCH = EPT // CH
RPT = NPAD // NS
RB = CH
RCH = RPT // RB

_PI = np.empty(128, np.int64)
for _c in range(4):
    for _i in range(16):
        _PI[32 * _c + _i] = 32 * _c + 2 * _i
        _PI[32 * _c + 16 + _i] = 32 * _c + 2 * _i + 1
_INV = np.argsort(_PI)


def _scatter_body(g_hbm, row_hbm, col_hbm, zeros_hbm, out_hbm,
                  ridx, cidx, bb0, bb1, fb0, fb1, acc, g0, g1, s0, s1):
    cid = lax.axis_index("c")
    sid = lax.axis_index("s")
    wid = sid * NC + cid
    pltpu.sync_copy(row_hbm.at[pl.ds(wid * NCH, NCH)], ridx)
    pltpu.sync_copy(col_hbm.at[pl.ds(wid * NCH, NCH)], cidx)
    pltpu.sync_copy(zeros_hbm, fb0)
    for k in range(RCH):
        pltpu.sync_copy(fb0, acc.at[pl.ds(sid * RPT + k * RB, RB)])
    plsc.subcore_barrier()

    bbs = (bb0, bb1)
    fbs = (fb0, fb1)
    gsem = (g0, g1)
    ssem = (s0, s1)
    d = fb0.shape[1]
    ngr = d // 32
    pltpu.async_copy(g_hbm.at[ridx.at[0]], bbs[0], gsem[0])
    niter = NCH // 2

    def step(i, carry):
        for k in range(2):
            j = 2 * i + k
            kn = 1 - k

            @pl.when(j + 1 < NCH)
            def _():
                pltpu.async_copy(g_hbm.at[ridx.at[j + 1]], bbs[kn], gsem[kn])

            pltpu.make_async_copy(g_hbm.at[ridx.at[j]], bbs[k], gsem[k]).wait()

            @pl.when(j >= 2)
            def _():
                pltpu.make_async_copy(
                    fbs[k], acc.at[cidx.at[j - 2]], ssem[k]).wait()

            def conv(r2, carry2):
                for u in range(2):
                    r = 2 * r2 + u
                    for c in range(ngr):
                        v = bbs[k][r, pl.ds(c * 32, 32)]
                        a, b = plsc.unpack(
                            v, format=plsc.PackFormat.INTERLEAVED)
                        fbs[k][r, pl.ds(c * 32, 16)] = a
                        fbs[k][r, pl.ds(c * 32 + 16, 16)] = b
                return carry2

            lax.fori_loop(0, CH // 2, conv, 0)
            pltpu.async_copy(fbs[k], acc.at[cidx.at[j]], ssem[k], add=True)
        return carry

    lax.fori_loop(0, niter, step, 0)
    pltpu.make_async_copy(fbs[0], acc.at[cidx.at[NCH - 2]], ssem[0]).wait()
    pltpu.make_async_copy(fbs[1], acc.at[cidx.at[NCH - 1]], ssem[1]).wait()
    plsc.subcore_barrier()
    for k in range(RCH):
        pltpu.sync_copy(acc.at[pl.ds(sid * RPT + k * RB, RB)], fb0)
        pltpu.sync_copy(fb0, out_hbm.at[cid, pl.ds(sid * RPT + k * RB, RB)])


def _deg_body(col_hbm, ones_hbm, zeros_hbm, out_hbm, cidx, ones_v, zbuf, acc,
              d0, d1):
    cid = lax.axis_index("c")
    sid = lax.axis_index("s")
    wid = sid * NC + cid
    pltpu.sync_copy(col_hbm.at[pl.ds(wid * NCH, NCH)], cidx)
    pltpu.sync_copy(ones_hbm, ones_v)
    pltpu.sync_copy(zeros_hbm, zbuf)
    pltpu.sync_copy(zbuf, acc.at[pl.ds(sid * RPT, RPT)])
    plsc.subcore_barrier()

    sems = (d0, d1)

    def step(i, carry):
        for k in range(2):
            j = 2 * i + k

            @pl.when(j >= 2)
            def _():
                pltpu.make_async_copy(
                    ones_v, acc.at[cidx.at[j - 2]], sems[k]).wait()

            pltpu.async_copy(ones_v, acc.at[cidx.at[j]], sems[k], add=True)
        return carry

    lax.fori_loop(0, NCH // 2, step, 0)
    for k in range(2):
        pltpu.make_async_copy(
            ones_v, acc.at[cidx.at[NCH - 2 + k]], sems[k]).wait()
    plsc.subcore_barrier()
    pltpu.sync_copy(acc.at[pl.ds(sid * RPT, RPT)], zbuf)
    pltpu.sync_copy(zbuf, out_hbm.at[cid, pl.ds(sid * RPT, RPT)])


@functools.lru_cache(maxsize=None)
def _make_deg():
    mesh = plsc.VectorSubcoreMesh(core_axis_name="c", subcore_axis_name="s")
    return pl.kernel(
        _deg_body,
        out_type=jax.ShapeDtypeStruct((NC, NPAD), jnp.float32),
        mesh=mesh,
        scratch_types=[
            pltpu.VMEM((NCH, CH), jnp.int32),
            pltpu.VMEM((CH,), jnp.float32),
            pltpu.VMEM((RPT,), jnp.float32),
            pltpu.VMEM_SHARED((NPAD,), jnp.float32),
        ] + [pltpu.SemaphoreType.DMA] * 2,
        compiler_params=pltpu.CompilerParams(use_tc_tiling_on_sc=False),
    )


@functools.lru_cache(maxsize=None)
def _make_scatter(d):
    mesh = plsc.VectorSubcoreMesh(core_axis_name="c", subcore_axis_name="s")
    return pl.kernel(
        _scatter_body,
        out_type=jax.ShapeDtypeStruct((NC, NPAD, d), jnp.float32),
        mesh=mesh,
        scratch_types=[
            pltpu.VMEM((NCH, CH), jnp.int32),
            pltpu.VMEM((NCH, CH), jnp.int32),
            pltpu.VMEM((CH, d), jnp.bfloat16),
            pltpu.VMEM((CH, d), jnp.bfloat16),
            pltpu.VMEM((CH, d), jnp.float32),
            pltpu.VMEM((CH, d), jnp.float32),
            pltpu.VMEM_SHARED((NPAD, d), jnp.float32),
        ] + [pltpu.SemaphoreType.DMA] * 4,
        compiler_params=pltpu.CompilerParams(use_tc_tiling_on_sc=False,
                                             needs_layout_passes=False),
    )


_BR = 1024
_NB = NPAD // _BR


def _tc_first_body(d0_ref, d1_ref, x_ref, w_ref, dis_ref, g_ref):
    deg = d0_ref[...] + d1_ref[...]
    dis = lax.rsqrt(deg)
    h = jnp.dot(x_ref[...], w_ref[...], preferred_element_type=jnp.float32)
    dis_ref[...] = dis
    g_ref[...] = (h * dis).astype(jnp.bfloat16)


def _tc_mid_body(a0_ref, a1_ref, dis_ref, b_ref, w_ref, g_ref):
    dis = dis_ref[...]
    act = dis * (a0_ref[...] + a1_ref[...]) + b_ref[...]
    act = jnp.maximum(act, 0.0)
    h = jnp.dot(act, w_ref[...], preferred_element_type=jnp.float32)
    g_ref[...] = (h * dis).astype(jnp.bfloat16)


def _tc_last_body(a0_ref, a1_ref, dis_ref, b_ref, out_ref):
    dis = dis_ref[...]
    out_ref[...] = dis * (a0_ref[...] + a1_ref[...]) + b_ref[...]


def _row_spec(d):
    return pl.BlockSpec((_BR, d), lambda i: (i, 0))


def _full_spec(r, c):
    return pl.BlockSpec((r, c), lambda i: (0, 0))


def _tc_first(deg0, deg1, x, w):
    din, dout = w.shape
    return pl.pallas_call(
        _tc_first_body,
        grid=(_NB,),
        in_specs=[_row_spec(1), _row_spec(1), _row_spec(din),
                  _full_spec(din, dout)],
        out_specs=[_row_spec(1), _row_spec(dout)],
        out_shape=[jax.ShapeDtypeStruct((NPAD, 1), jnp.float32),
                   jax.ShapeDtypeStruct((NPAD, dout), jnp.bfloat16)],
    )(deg0, deg1, x, w)


def _tc_mid(a0, a1, dis, b, w):
    din, dout = w.shape
    return pl.pallas_call(
        _tc_mid_body,
        grid=(_NB,),
        in_specs=[_row_spec(din), _row_spec(din), _row_spec(1),
                  _full_spec(1, din), _full_spec(din, dout)],
        out_specs=_row_spec(dout),
        out_shape=jax.ShapeDtypeStruct((NPAD, dout), jnp.bfloat16),
    )(a0, a1, dis, b, w)


def _tc_last(a0, a1, dis, b):
    d = a0.shape[1]
    return pl.pallas_call(
        _tc_last_body,
        grid=(_NB,),
        in_specs=[_row_spec(d), _row_spec(d), _row_spec(1),
                  _full_spec(1, d)],
        out_specs=_row_spec(d),
        out_shape=jax.ShapeDtypeStruct((NPAD, d), jnp.float32),
    )(a0, a1, dis, b)


def kernel(x, edge_index, W1, b1, W2, b2, W3, b3, W4, b4, W5, b5):
    loop = jnp.arange(N, dtype=edge_index.dtype)
    rowp = jnp.pad(jnp.concatenate([edge_index[0], loop]), (0, EPAD - EL))
    colp = jnp.pad(jnp.concatenate([edge_index[1], loop]), (0, EPAD - EL),
                   constant_values=N)
    row2 = rowp.reshape(EPAD // CH, CH)
    col2 = colp.reshape(EPAD // CH, CH)
    xp = jnp.pad(x, ((0, NPAD - N), (0, 0)))

    degp = _make_deg()(col2, jnp.ones((CH,), jnp.float32),
                       jnp.zeros((RPT,), jnp.float32))
    dis, g = _tc_first(degp[0][:, None], degp[1][:, None], xp, W1)

    pi = jnp.asarray(_PI)
    w5p = jnp.pad(W5, ((0, 0), (0, 128 - W5.shape[1])))
    b5p = jnp.pad(b5, (0, 128 - b5.shape[0]))
    ws = [W2[pi], W3[pi], W4[pi], w5p[pi]]
    bs = [b1[pi], b2[pi], b3[pi], b4[pi]]
    z128 = jnp.zeros((RB, 128), jnp.float32)

    for i in range(4):
        aggp = _make_scatter(128)(g, row2, col2, z128)
        g = _tc_mid(aggp[0], aggp[1], dis, bs[i].reshape(1, -1), ws[i])
    aggp = _make_scatter(128)(g, row2, col2, z128)
    out = _tc_last(aggp[0], aggp[1], dis, b5p[pi].reshape(1, -1))
    return out[:N][:, jnp.asarray(_INV[:64])]


# speedup vs baseline: 1.1861x; 1.0080x over previous
import functools

import jax
import jax.numpy as jnp
import numpy as np
from jax import lax
from jax.experimental import pallas as pl
from jax.experimental.pallas import tpu as pltpu
from jax.experimental.pallas import tpu_sc as plsc

N = 10000
NPAD = 10240
E = 320000
EL = E + N
NC, NS = 2, 16
NW = NC * NS
CH = 64
EPAD = 331776
EPT = EPAD // NW
NCH = EPT // CH
RPT = NPAD // NS
RB = CH
RCH = RPT // RB

_PI = np.empty(128, np.int64)
for _c in range(4):
    for _i in range(16):
        _PI[32 * _c + _i] = 32 * _c + 2 * _i
        _PI[32 * _c + 16 + _i] = 32 * _c + 2 * _i + 1
_INV = np.argsort(_PI)


def _scatter_body(g_hbm, row_hbm, col_hbm, zeros_hbm, out_hbm,
                  ridx, cidx, bb0, bb1, fb0, fb1, acc, g0, g1, s0, s1):
    cid = lax.axis_index("c")
    sid = lax.axis_index("s")
    wid = sid * NC + cid
    pltpu.sync_copy(row_hbm.at[pl.ds(wid * NCH, NCH)], ridx)
    pltpu.sync_copy(col_hbm.at[pl.ds(wid * NCH, NCH)], cidx)
    pltpu.sync_copy(zeros_hbm, fb0)
    for k in range(RCH):
        pltpu.sync_copy(fb0, acc.at[pl.ds(sid * RPT + k * RB, RB)])
    plsc.subcore_barrier()

    bbs = (bb0, bb1)
    fbs = (fb0, fb1)
    gsem = (g0, g1)
    ssem = (s0, s1)
    d = fb0.shape[1]
    ngr = d // 32
    pltpu.async_copy(g_hbm.at[ridx.at[0]], bbs[0], gsem[0])
    niter = NCH // 2

    def step(i, carry):
        for k in range(2):
            j = 2 * i + k
            kn = 1 - k

            @pl.when(j + 1 < NCH)
            def _():
                pltpu.async_copy(g_hbm.at[ridx.at[j + 1]], bbs[kn], gsem[kn])

            pltpu.make_async_copy(g_hbm.at[ridx.at[j]], bbs[k], gsem[k]).wait()

            @pl.when(j >= 2)
            def _():
                pltpu.make_async_copy(
                    fbs[k], acc.at[cidx.at[j - 2]], ssem[k]).wait()

            def conv(r2, carry2):
                for u in range(2):
                    r = 2 * r2 + u
                    for c in range(ngr):
                        v = bbs[k][r, pl.ds(c * 32, 32)]
                        a, b = plsc.unpack(
                            v, format=plsc.PackFormat.INTERLEAVED)
                        fbs[k][r, pl.ds(c * 32, 16)] = a
                        fbs[k][r, pl.ds(c * 32 + 16, 16)] = b
                return carry2

            lax.fori_loop(0, CH // 2, conv, 0)
            pltpu.async_copy(fbs[k], acc.at[cidx.at[j]], ssem[k], add=True)
        return carry

    lax.fori_loop(0, niter, step, 0)
    pltpu.make_async_copy(fbs[0], acc.at[cidx.at[NCH - 2]], ssem[0]).wait()
    pltpu.make_async_copy(fbs[1], acc.at[cidx.at[NCH - 1]], ssem[1]).wait()
    plsc.subcore_barrier()
    for k in range(RCH):
        pltpu.sync_copy(acc.at[pl.ds(sid * RPT + k * RB, RB)], fb0)
        pltpu.sync_copy(fb0, out_hbm.at[cid, pl.ds(sid * RPT + k * RB, RB)])


def _deg_body(col_hbm, ones_hbm, zeros_hbm, out_hbm, cidx, ones_v, zbuf, acc,
              d0, d1):
    cid = lax.axis_index("c")
    sid = lax.axis_index("s")
    wid = sid * NC + cid
    pltpu.sync_copy(col_hbm.at[pl.ds(wid * NCH, NCH)], cidx)
    pltpu.sync_copy(ones_hbm, ones_v)
    pltpu.sync_copy(zeros_hbm, zbuf)
    pltpu.sync_copy(zbuf, acc.at[pl.ds(sid * RPT, RPT)])
    plsc.subcore_barrier()

    sems = (d0, d1)

    def step(i, carry):
        for k in range(2):
            j = 2 * i + k

            @pl.when(j >= 2)
            def _():
                pltpu.make_async_copy(
                    ones_v, acc.at[cidx.at[j - 2]], sems[k]).wait()

            pltpu.async_copy(ones_v, acc.at[cidx.at[j]], sems[k], add=True)
        return carry

    lax.fori_loop(0, NCH // 2, step, 0)
    for k in range(2):
        pltpu.make_async_copy(
            ones_v, acc.at[cidx.at[NCH - 2 + k]], sems[k]).wait()
    plsc.subcore_barrier()
    pltpu.sync_copy(acc.at[pl.ds(sid * RPT, RPT)], zbuf)
    pltpu.sync_copy(zbuf, out_hbm.at[cid, pl.ds(sid * RPT, RPT)])


@functools.lru_cache(maxsize=None)
def _make_deg():
    mesh = plsc.VectorSubcoreMesh(core_axis_name="c", subcore_axis_name="s")
    return pl.kernel(
        _deg_body,
        out_type=jax.ShapeDtypeStruct((NC, NPAD), jnp.float32),
        mesh=mesh,
        scratch_types=[
            pltpu.VMEM((NCH, CH), jnp.int32),
            pltpu.VMEM((CH,), jnp.float32),
            pltpu.VMEM((RPT,), jnp.float32),
            pltpu.VMEM_SHARED((NPAD,), jnp.float32),
        ] + [pltpu.SemaphoreType.DMA] * 2,
        compiler_params=pltpu.CompilerParams(use_tc_tiling_on_sc=False),
    )


@functools.lru_cache(maxsize=None)
def _make_scatter(d):
    mesh = plsc.VectorSubcoreMesh(core_axis_name="c", subcore_axis_name="s")
    return pl.kernel(
        _scatter_body,
        out_type=jax.ShapeDtypeStruct((NC, NPAD, d), jnp.float32),
        mesh=mesh,
        scratch_types=[
            pltpu.VMEM((NCH, CH), jnp.int32),
            pltpu.VMEM((NCH, CH), jnp.int32),
            pltpu.VMEM((CH, d), jnp.bfloat16),
            pltpu.VMEM((CH, d), jnp.bfloat16),
            pltpu.VMEM((CH, d), jnp.float32),
            pltpu.VMEM((CH, d), jnp.float32),
            pltpu.VMEM_SHARED((NPAD, d), jnp.float32),
        ] + [pltpu.SemaphoreType.DMA] * 4,
        compiler_params=pltpu.CompilerParams(use_tc_tiling_on_sc=False,
                                             needs_layout_passes=False),
    )


_BR = 1024
_NB = NPAD // _BR


def _tc_first_body(dp_ref, x_ref, w_ref, dis_ref, g_ref):
    deg = dp_ref[0] + dp_ref[1]
    dis = lax.rsqrt(deg)
    h = jnp.dot(x_ref[...], w_ref[...], preferred_element_type=jnp.float32)
    dis_ref[...] = dis
    g_ref[...] = (h * dis).astype(jnp.bfloat16)


def _tc_mid_body(agg_ref, dis_ref, b_ref, w_ref, g_ref):
    dis = dis_ref[...]
    act = dis * (agg_ref[0] + agg_ref[1]) + b_ref[...]
    act = jnp.maximum(act, 0.0)
    h = jnp.dot(act, w_ref[...], preferred_element_type=jnp.float32)
    g_ref[...] = (h * dis).astype(jnp.bfloat16)


def _tc_last_body(agg_ref, dis_ref, b_ref, out_ref):
    dis = dis_ref[...]
    out_ref[...] = dis * (agg_ref[0] + agg_ref[1]) + b_ref[...]


def _row_spec(d):
    return pl.BlockSpec((_BR, d), lambda i: (i, 0))


def _full_spec(r, c):
    return pl.BlockSpec((r, c), lambda i: (0, 0))


def _tc_first(degp, x, w):
    din, dout = w.shape
    return pl.pallas_call(
        _tc_first_body,
        grid=(_NB,),
        in_specs=[_agg_spec(1), _row_spec(din), _full_spec(din, dout)],
        out_specs=[_row_spec(1), _row_spec(dout)],
        out_shape=[jax.ShapeDtypeStruct((NPAD, 1), jnp.float32),
                   jax.ShapeDtypeStruct((NPAD, dout), jnp.bfloat16)],
    )(degp, x, w)


def _agg_spec(d):
    return pl.BlockSpec((NC, _BR, d), lambda i: (0, i, 0))


def _tc_mid(aggp, dis, b, w):
    din, dout = w.shape
    return pl.pallas_call(
        _tc_mid_body,
        grid=(_NB,),
        in_specs=[_agg_spec(din), _row_spec(1),
                  _full_spec(1, din), _full_spec(din, dout)],
        out_specs=_row_spec(dout),
        out_shape=jax.ShapeDtypeStruct((NPAD, dout), jnp.bfloat16),
    )(aggp, dis, b, w)


def _tc_last(aggp, dis, b):
    d = aggp.shape[2]
    return pl.pallas_call(
        _tc_last_body,
        grid=(_NB,),
        in_specs=[_agg_spec(d), _row_spec(1), _full_spec(1, d)],
        out_specs=_row_spec(d),
        out_shape=jax.ShapeDtypeStruct((NPAD, d), jnp.float32),
    )(aggp, dis, b)


def kernel(x, edge_index, W1, b1, W2, b2, W3, b3, W4, b4, W5, b5):
    loop = jnp.arange(N, dtype=edge_index.dtype)
    rowp = jnp.pad(jnp.concatenate([edge_index[0], loop]), (0, EPAD - EL))
    colp = jnp.pad(jnp.concatenate([edge_index[1], loop]), (0, EPAD - EL),
                   constant_values=N)
    row2 = rowp.reshape(EPAD // CH, CH)
    col2 = colp.reshape(EPAD // CH, CH)
    xp = jnp.pad(x, ((0, NPAD - N), (0, 0)))

    degp = _make_deg()(col2, jnp.ones((CH,), jnp.float32),
                       jnp.zeros((RPT,), jnp.float32))
    dis, g = _tc_first(degp[..., None], xp, W1)

    pi = jnp.asarray(_PI)
    w5p = jnp.pad(W5, ((0, 0), (0, 128 - W5.shape[1])))
    b5p = jnp.pad(b5, (0, 128 - b5.shape[0]))
    ws = [W2[pi], W3[pi], W4[pi], w5p[pi]]
    bs = [b1[pi], b2[pi], b3[pi], b4[pi]]
    z128 = jnp.zeros((RB, 128), jnp.float32)

    for i in range(4):
        aggp = _make_scatter(128)(g, row2, col2, z128)
        g = _tc_mid(aggp, dis, bs[i].reshape(1, -1), ws[i])
    aggp = _make_scatter(128)(g, row2, col2, z128)
    out = _tc_last(aggp, dis, b5p[pi].reshape(1, -1))
    return out[:N][:, jnp.asarray(_INV[:64])]


# speedup vs baseline: 1.1875x; 1.0012x over previous
import functools

import jax
import jax.numpy as jnp
import numpy as np
from jax import lax
from jax.experimental import pallas as pl
from jax.experimental.pallas import tpu as pltpu
from jax.experimental.pallas import tpu_sc as plsc

N = 10000
NPAD = 10240
E = 320000
EL = E + N
NC, NS = 2, 16
NW = NC * NS
CH = 64
EPAD = 331776
EPT = EPAD // NW
NCH = EPT // CH
RPT = NPAD // NS
RB = CH
RCH = RPT // RB

_PI = np.empty(128, np.int64)
for _c in range(4):
    for _i in range(16):
        _PI[32 * _c + _i] = 32 * _c + 2 * _i
        _PI[32 * _c + 16 + _i] = 32 * _c + 2 * _i + 1
_INV = np.argsort(_PI)


def _scatter_body(g_hbm, row_hbm, col_hbm, zeros_hbm, out_hbm,
                  ridx, cidx, bb0, bb1, fb0, fb1, acc, g0, g1, s0, s1):
    cid = lax.axis_index("c")
    sid = lax.axis_index("s")
    wid = sid * NC + cid
    pltpu.sync_copy(row_hbm.at[pl.ds(wid * NCH, NCH)], ridx)
    pltpu.sync_copy(col_hbm.at[pl.ds(wid * NCH, NCH)], cidx)
    pltpu.sync_copy(zeros_hbm, fb0)
    for k in range(RCH):
        pltpu.sync_copy(fb0, acc.at[pl.ds(sid * RPT + k * RB, RB)])
    plsc.subcore_barrier()

    bbs = (bb0, bb1)
    fbs = (fb0, fb1)
    gsem = (g0, g1)
    ssem = (s0, s1)
    d = fb0.shape[1]
    ngr = d // 32
    pltpu.async_copy(g_hbm.at[ridx.at[0]], bbs[0], gsem[0])
    niter = NCH // 2

    def step(i, carry):
        for k in range(2):
            j = 2 * i + k
            kn = 1 - k

            @pl.when(j + 1 < NCH)
            def _():
                pltpu.async_copy(g_hbm.at[ridx.at[j + 1]], bbs[kn], gsem[kn])

            pltpu.make_async_copy(g_hbm.at[ridx.at[j]], bbs[k], gsem[k]).wait()

            @pl.when(j >= 2)
            def _():
                pltpu.make_async_copy(
                    fbs[k], acc.at[cidx.at[j - 2]], ssem[k]).wait()

            def conv(r2, carry2):
                for u in range(2):
                    r = 2 * r2 + u
                    for c in range(ngr):
                        v = bbs[k][r, pl.ds(c * 32, 32)]
                        a, b = plsc.unpack(
                            v, format=plsc.PackFormat.INTERLEAVED)
                        fbs[k][r, pl.ds(c * 32, 16)] = a
                        fbs[k][r, pl.ds(c * 32 + 16, 16)] = b
                return carry2

            lax.fori_loop(0, CH // 2, conv, 0)
            pltpu.async_copy(fbs[k], acc.at[cidx.at[j]], ssem[k], add=True)
        return carry

    lax.fori_loop(0, niter, step, 0)
    pltpu.make_async_copy(fbs[0], acc.at[cidx.at[NCH - 2]], ssem[0]).wait()
    pltpu.make_async_copy(fbs[1], acc.at[cidx.at[NCH - 1]], ssem[1]).wait()
    plsc.subcore_barrier()
    for k in range(RCH):
        pltpu.sync_copy(acc.at[pl.ds(sid * RPT + k * RB, RB)], fb0)
        pltpu.sync_copy(fb0, out_hbm.at[cid, pl.ds(sid * RPT + k * RB, RB)])


def _deg_body(col_hbm, ones_hbm, zeros_hbm, out_hbm, cidx, ones_v, zbuf, acc,
              d0, d1):
    cid = lax.axis_index("c")
    sid = lax.axis_index("s")
    wid = sid * NC + cid
    pltpu.sync_copy(col_hbm.at[pl.ds(wid * NCH, NCH)], cidx)
    pltpu.sync_copy(ones_hbm, ones_v)
    pltpu.sync_copy(zeros_hbm, zbuf)
    pltpu.sync_copy(zbuf, acc.at[pl.ds(sid * RPT, RPT)])
    plsc.subcore_barrier()

    sems = (d0, d1)

    def step(i, carry):
        for k in range(2):
            j = 2 * i + k

            @pl.when(j >= 2)
            def _():
                pltpu.make_async_copy(
                    ones_v, acc.at[cidx.at[j - 2]], sems[k]).wait()

            pltpu.async_copy(ones_v, acc.at[cidx.at[j]], sems[k], add=True)
        return carry

    lax.fori_loop(0, NCH // 2, step, 0)
    for k in range(2):
        pltpu.make_async_copy(
            ones_v, acc.at[cidx.at[NCH - 2 + k]], sems[k]).wait()
    plsc.subcore_barrier()
    pltpu.sync_copy(acc.at[pl.ds(sid * RPT, RPT)], zbuf)
    pltpu.sync_copy(zbuf, out_hbm.at[cid, pl.ds(sid * RPT, RPT)])


@functools.lru_cache(maxsize=None)
def _make_deg():
    mesh = plsc.VectorSubcoreMesh(core_axis_name="c", subcore_axis_name="s")
    return pl.kernel(
        _deg_body,
        out_type=jax.ShapeDtypeStruct((NC, NPAD), jnp.float32),
        mesh=mesh,
        scratch_types=[
            pltpu.VMEM((NCH, CH), jnp.int32),
            pltpu.VMEM((CH,), jnp.float32),
            pltpu.VMEM((RPT,), jnp.float32),
            pltpu.VMEM_SHARED((NPAD,), jnp.float32),
        ] + [pltpu.SemaphoreType.DMA] * 2,
        compiler_params=pltpu.CompilerParams(use_tc_tiling_on_sc=False),
    )


@functools.lru_cache(maxsize=None)
def _make_scatter(d):
    mesh = plsc.VectorSubcoreMesh(core_axis_name="c", subcore_axis_name="s")
    return pl.kernel(
        _scatter_body,
        out_type=jax.ShapeDtypeStruct((NC, NPAD, d), jnp.float32),
        mesh=mesh,
        scratch_types=[
            pltpu.VMEM((NCH, CH), jnp.int32),
            pltpu.VMEM((NCH, CH), jnp.int32),
            pltpu.VMEM((CH, d), jnp.bfloat16),
            pltpu.VMEM((CH, d), jnp.bfloat16),
            pltpu.VMEM((CH, d), jnp.float32),
            pltpu.VMEM((CH, d), jnp.float32),
            pltpu.VMEM_SHARED((NPAD, d), jnp.float32),
        ] + [pltpu.SemaphoreType.DMA] * 4,
        compiler_params=pltpu.CompilerParams(use_tc_tiling_on_sc=False,
                                             needs_layout_passes=False),
    )


_BR = 1024
_NB = NPAD // _BR


def _tc_first_body(dp_ref, x_ref, w_ref, dis_ref, g_ref):
    deg = dp_ref[0] + dp_ref[1]
    dis = lax.rsqrt(deg)
    h = jnp.dot(x_ref[...], w_ref[...], preferred_element_type=jnp.float32)
    dis_ref[...] = dis
    g_ref[...] = (h * dis).astype(jnp.bfloat16).reshape(_BR * 128)


def _tc_mid_body(agg_ref, dis_ref, b_ref, w_ref, g_ref):
    dis = dis_ref[...]
    act = dis * (agg_ref[0] + agg_ref[1]) + b_ref[...]
    act = jnp.maximum(act, 0.0)
    h = jnp.dot(act, w_ref[...], preferred_element_type=jnp.float32)
    g_ref[...] = (h * dis).astype(jnp.bfloat16).reshape(_BR * 128)


def _tc_last_body(agg_ref, dis_ref, b_ref, out_ref):
    dis = dis_ref[...]
    out_ref[...] = dis * (agg_ref[0] + agg_ref[1]) + b_ref[...]


def _row_spec(d):
    return pl.BlockSpec((_BR, d), lambda i: (i, 0))


def _full_spec(r, c):
    return pl.BlockSpec((r, c), lambda i: (0, 0))


def _tc_first(degp, x, w):
    din, dout = w.shape
    return pl.pallas_call(
        _tc_first_body,
        grid=(_NB,),
        in_specs=[_agg_spec(1), _row_spec(din), _full_spec(din, dout)],
        out_specs=[_row_spec(1), pl.BlockSpec((_BR * dout,), lambda i: (i,))],
        out_shape=[jax.ShapeDtypeStruct((NPAD, 1), jnp.float32),
                   jax.ShapeDtypeStruct((NPAD * dout,), jnp.bfloat16)],
    )(degp, x, w)


def _agg_spec(d):
    return pl.BlockSpec((NC, _BR, d), lambda i: (0, i, 0))


def _tc_mid(aggp, dis, b, w):
    din, dout = w.shape
    return pl.pallas_call(
        _tc_mid_body,
        grid=(_NB,),
        in_specs=[_agg_spec(din), _row_spec(1),
                  _full_spec(1, din), _full_spec(din, dout)],
        out_specs=pl.BlockSpec((_BR * dout,), lambda i: (i,)),
        out_shape=jax.ShapeDtypeStruct((NPAD * dout,), jnp.bfloat16),
    )(aggp, dis, b, w)


def _tc_last(aggp, dis, b):
    d = aggp.shape[2]
    return pl.pallas_call(
        _tc_last_body,
        grid=(_NB,),
        in_specs=[_agg_spec(d), _row_spec(1), _full_spec(1, d)],
        out_specs=_row_spec(d),
        out_shape=jax.ShapeDtypeStruct((NPAD, d), jnp.float32),
    )(aggp, dis, b)


def kernel(x, edge_index, W1, b1, W2, b2, W3, b3, W4, b4, W5, b5):
    loop = jnp.arange(N, dtype=edge_index.dtype)
    rowp = jnp.pad(jnp.concatenate([edge_index[0], loop]), (0, EPAD - EL))
    colp = jnp.pad(jnp.concatenate([edge_index[1], loop]), (0, EPAD - EL),
                   constant_values=N)
    row2 = rowp.reshape(EPAD // CH, CH)
    col2 = colp.reshape(EPAD // CH, CH)
    xp = jnp.pad(x, ((0, NPAD - N), (0, 0)))

    degp = _make_deg()(col2, jnp.ones((CH,), jnp.float32),
                       jnp.zeros((RPT,), jnp.float32))
    dis, g = _tc_first(degp[..., None], xp, W1)

    pi = jnp.asarray(_PI)
    w5p = jnp.pad(W5, ((0, 0), (0, 128 - W5.shape[1])))
    b5p = jnp.pad(b5, (0, 128 - b5.shape[0]))
    ws = [W2[pi], W3[pi], W4[pi], w5p[pi]]
    bs = [b1[pi], b2[pi], b3[pi], b4[pi]]
    z128 = jnp.zeros((RB, 128), jnp.float32)

    for i in range(4):
        aggp = _make_scatter(128)(g.reshape(NPAD, 128), row2, col2, z128)
        g = _tc_mid(aggp, dis, bs[i].reshape(1, -1), ws[i])
    aggp = _make_scatter(128)(g.reshape(NPAD, 128), row2, col2, z128)
    out = _tc_last(aggp, dis, b5p[pi].reshape(1, -1))
    return out[:N][:, jnp.asarray(_INV[:64])]
